# baseline (speedup 1.0000x reference)
import jax
import jax.numpy as jnp
from jax import lax
from jax.experimental import pallas as pl
from jax.experimental.pallas import tpu as pltpu

_NUM_CLASSES = 20
_THRESHOLD = 0.5
_NEGPOS_RATIO = 3
_VAR0 = 0.1
_VAR1 = 0.2
_BATCH = 32
_NUM_PRIORS = 32768
_NUM_OBJS = 8
_R = _NUM_PRIORS // 128


def _mbox_a1_kernel(priors_ref, targets_ref, loc_ref, obj_ref,
                    mined_ref, conft_ref, stats_ref):
    f32 = jnp.float32
    pcx = priors_ref[0]
    pcy = priors_ref[1]
    pw = priors_ref[2]
    ph = priors_ref[3]
    px1 = pcx - pw * 0.5
    py1 = pcy - ph * 0.5
    px2 = pcx + pw * 0.5
    py2 = pcy + ph * 0.5
    area_p = (px2 - px1) * (py2 - py1)

    fi = (lax.broadcasted_iota(jnp.int32, (_R, 128), 0) * 128
          + lax.broadcasted_iota(jnp.int32, (_R, 128), 1))

    tx1 = [targets_ref[0, j, 0] for j in range(_NUM_OBJS)]
    ty1 = [targets_ref[0, j, 1] for j in range(_NUM_OBJS)]
    tx2 = [targets_ref[0, j, 2] for j in range(_NUM_OBJS)]
    ty2 = [targets_ref[0, j, 3] for j in range(_NUM_OBJS)]
    tlab = [targets_ref[0, j, 4] for j in range(_NUM_OBJS)]

    bto = None
    bti = None
    bpi = []
    for j in range(_NUM_OBJS):
        ix = jnp.maximum(jnp.minimum(tx2[j], px2) - jnp.maximum(tx1[j], px1), 0.0)
        iy = jnp.maximum(jnp.minimum(ty2[j], py2) - jnp.maximum(ty1[j], py1), 0.0)
        inter = ix * iy
        area_t = (tx2[j] - tx1[j]) * (ty2[j] - ty1[j])
        ov = inter / (area_t + area_p - inter)
        if j == 0:
            bto = ov
            bti = jnp.zeros((_R, 128), jnp.int32)
        else:
            upd = ov > bto
            bti = jnp.where(upd, jnp.int32(j), bti)
            bto = jnp.where(upd, ov, bto)
        m = jnp.max(ov)
        bpi.append(jnp.min(jnp.where(ov == m, fi, jnp.int32(2 ** 30))))

    for j in range(_NUM_OBJS):
        hit = fi == bpi[j]
        bto = jnp.where(hit, 2.0, bto)
        bti = jnp.where(hit, jnp.int32(j), bti)

    conf_t = jnp.where(bti == 0, tlab[0].astype(jnp.int32), 0)
    mcx = jnp.where(bti == 0, (tx1[0] + tx2[0]) * 0.5, 0.0)
    mcy = jnp.where(bti == 0, (ty1[0] + ty2[0]) * 0.5, 0.0)
    mw = jnp.where(bti == 0, tx2[0] - tx1[0], 0.0)
    mh = jnp.where(bti == 0, ty2[0] - ty1[0], 0.0)
    for j in range(1, _NUM_OBJS):
        sel = bti == j
        conf_t = jnp.where(sel, tlab[j].astype(jnp.int32), conf_t)
        mcx = jnp.where(sel, (tx1[j] + tx2[j]) * 0.5, mcx)
        mcy = jnp.where(sel, (ty1[j] + ty2[j]) * 0.5, mcy)
        mw = jnp.where(sel, tx2[j] - tx1[j], mw)
        mh = jnp.where(sel, ty2[j] - ty1[j], mh)
    conf_t = jnp.where(bto < _THRESHOLD, 0, conf_t)
    conft_ref[0] = conf_t
    pos = conf_t > 0
    posf = pos.astype(f32)
    num_pos = jnp.sum(conf_t > 0, dtype=jnp.int32)

    g_cx = (mcx - pcx) / (_VAR0 * pw)
    g_cy = (mcy - pcy) / (_VAR0 * ph)
    g_w = jnp.log(mw / pw) / _VAR1
    g_h = jnp.log(mh / ph) / _VAR1
    loss_l = jnp.float32(0.0)
    for c, g in enumerate((g_cx, g_cy, g_w, g_h)):
        d = loc_ref[0, c] - g
        ad = jnp.abs(d)
        sl1 = jnp.where(ad < 1.0, 0.5 * d * d, ad - 0.5)
        loss_l = loss_l + jnp.sum(sl1 * posf)

    o0 = obj_ref[0, 0]
    o1 = obj_ref[0, 1]
    mo = jnp.maximum(o0, o1)
    lse_o = mo + jnp.log(jnp.exp(o0 - mo) + jnp.exp(o1 - mo))
    mined_ref[0] = jnp.where(pos, 0.0, lse_o - o0)
    sum_b = jnp.sum(posf * (lse_o - o1))

    lane = lax.broadcasted_iota(jnp.int32, (1, 8), 1)
    stats_ref[0] = (jnp.where(lane == 0, loss_l, 0.0)
                    + jnp.where(lane == 2, sum_b, 0.0)
                    + jnp.where(lane == 3, num_pos.astype(f32), 0.0))


def _mbox_a2_kernel(conf_ref, conft_ref, stats2_ref):
    f32 = jnp.float32
    conf_t = conft_ref[0]
    pos = conf_t > 0
    posf = pos.astype(f32)
    cf = [conf_ref[0, c].astype(f32) for c in range(_NUM_CLASSES)]
    s = jnp.exp(cf[0])
    for c in range(1, _NUM_CLASSES):
        s = s + jnp.exp(cf[c])
    lse_c = jnp.log(s)
    c_sel = jnp.zeros((_R, 128), f32)
    for c in range(_NUM_CLASSES):
        c_sel = jnp.where(conf_t == c + 1, cf[c], c_sel)
    sum_ac = jnp.sum(posf * (lse_c - c_sel))
    lane = lax.broadcasted_iota(jnp.int32, (1, 8), 1)
    stats2_ref[0] = jnp.where(lane == 1, sum_ac, 0.0)


def _mbox_b_kernel(mined_ref, stats_ref, stats2_ref, out_ref):
    vals = mined_ref[...]
    keys = lax.bitcast_convert_type(vals, jnp.int32)
    np_f = stats_ref[:, 3:4]
    k = jnp.minimum(jnp.int32(_NEGPOS_RATIO) * np_f.astype(jnp.int32),
                    jnp.int32(_NUM_PRIORS - 1))
    lo = jnp.zeros((_BATCH, 1), jnp.int32)
    for bit in range(30, -1, -1):
        cand = jnp.bitwise_or(lo, jnp.int32(1 << bit))
        cnt = jnp.sum((keys >= cand).astype(jnp.int32), axis=1, keepdims=True)
        lo = jnp.where(cnt >= k, cand, lo)
    gt = keys > lo
    cnt_gt = jnp.sum(gt.astype(jnp.int32), axis=1, keepdims=True)
    sum_gt = jnp.sum(jnp.where(gt, vals, 0.0), axis=1, keepdims=True)
    t_f = lax.bitcast_convert_type(lo, jnp.float32)
    tk = sum_gt + jnp.where(k > 0, (k - cnt_gt).astype(jnp.float32) * t_f, 0.0)

    loss_l = jnp.sum(stats_ref[:, 0])
    loss_c = jnp.sum(stats2_ref[:, 1] + stats_ref[:, 2] + tk[:, 0])
    loss_o = jnp.sum(stats_ref[:, 2] + tk[:, 0])
    n = jnp.sum(stats_ref[:, 3])
    lane = lax.broadcasted_iota(jnp.int32, (1, 8), 1)
    out_ref[...] = (jnp.where(lane == 0, loss_l, 0.0)
                    + jnp.where(lane == 1, loss_c, 0.0)
                    + jnp.where(lane == 2, loss_o, 0.0)
                    + jnp.where(lane == 3, n, 0.0))


@jax.jit
def _run(loc_s, conf_s, obj_s, priors_s, targets):
    mined, conft, stats = pl.pallas_call(
        _mbox_a1_kernel,
        grid=(_BATCH,),
        in_specs=[
            pl.BlockSpec((4, _R, 128), lambda b: (0, 0, 0)),
            pl.BlockSpec((1, _NUM_OBJS, 5), lambda b: (b, 0, 0)),
            pl.BlockSpec((1, 4, _R, 128), lambda b: (b, 0, 0, 0)),
            pl.BlockSpec((1, 2, _R, 128), lambda b: (b, 0, 0, 0)),
        ],
        out_specs=[
            pl.BlockSpec((1, _R, 128), lambda b: (b, 0, 0)),
            pl.BlockSpec((1, _R, 128), lambda b: (b, 0, 0)),
            pl.BlockSpec((1, 1, 8), lambda b: (b, 0, 0)),
        ],
        out_shape=[
            jax.ShapeDtypeStruct((_BATCH, _R, 128), jnp.float32),
            jax.ShapeDtypeStruct((_BATCH, _R, 128), jnp.int32),
            jax.ShapeDtypeStruct((_BATCH, 1, 8), jnp.float32),
        ],
        compiler_params=pltpu.CompilerParams(
            dimension_semantics=("parallel",),
        ),
    )(priors_s, targets, loc_s, obj_s)

    stats2 = pl.pallas_call(
        _mbox_a2_kernel,
        grid=(_BATCH,),
        in_specs=[
            pl.BlockSpec((1, _NUM_CLASSES, _R, 128), lambda b: (b, 0, 0, 0)),
            pl.BlockSpec((1, _R, 128), lambda b: (b, 0, 0)),
        ],
        out_specs=pl.BlockSpec((1, 1, 8), lambda b: (b, 0, 0)),
        out_shape=jax.ShapeDtypeStruct((_BATCH, 1, 8), jnp.float32),
        compiler_params=pltpu.CompilerParams(
            dimension_semantics=("parallel",),
        ),
    )(conf_s, conft)

    out = pl.pallas_call(
        _mbox_b_kernel,
        grid=(1,),
        in_specs=[
            pl.BlockSpec((_BATCH, _NUM_PRIORS), lambda i: (0, 0)),
            pl.BlockSpec((_BATCH, 8), lambda i: (0, 0)),
            pl.BlockSpec((_BATCH, 8), lambda i: (0, 0)),
        ],
        out_specs=pl.BlockSpec((1, 8), lambda i: (0, 0)),
        out_shape=jax.ShapeDtypeStruct((1, 8), jnp.float32),
    )(mined.reshape(_BATCH, _NUM_PRIORS), stats.reshape(_BATCH, 8),
      stats2.reshape(_BATCH, 8))
    n = out[0, 3]
    return out[0, 0] / n, out[0, 1] / n, out[0, 2] / n


def kernel(loc_data, conf_data, obj_data, priors, targets):
    loc_s = loc_data.transpose(0, 2, 1).reshape(_BATCH, 4, _R, 128)
    conf_s = (conf_data.astype(jnp.bfloat16)
              .transpose(0, 2, 1).reshape(_BATCH, _NUM_CLASSES, _R, 128))
    obj_s = obj_data.transpose(0, 2, 1).reshape(_BATCH, 2, _R, 128)
    priors_s = priors.T.reshape(4, _R, 128)
    return _run(loc_s, conf_s, obj_s, priors_s, targets)


# speedup vs baseline: 1.0520x; 1.0520x over previous
import jax
import jax.numpy as jnp
from jax import lax
from jax.experimental import pallas as pl
from jax.experimental.pallas import tpu as pltpu

_NUM_CLASSES = 20
_THRESHOLD = 0.5
_NEGPOS_RATIO = 3
_VAR0 = 0.1
_VAR1 = 0.2
_BATCH = 32
_NUM_PRIORS = 32768
_NUM_OBJS = 8
_R = _NUM_PRIORS // 128


def _mbox_a_kernel(priors_ref, targets_ref, loc_ref, conf_ref, obj_ref,
                   mined_ref, stats_ref):
    f32 = jnp.float32
    pcx = priors_ref[0]
    pcy = priors_ref[1]
    pw = priors_ref[2]
    ph = priors_ref[3]
    px1 = pcx - pw * 0.5
    py1 = pcy - ph * 0.5
    px2 = pcx + pw * 0.5
    py2 = pcy + ph * 0.5
    area_p = (px2 - px1) * (py2 - py1)

    fi = (lax.broadcasted_iota(jnp.int32, (_R, 128), 0) * 128
          + lax.broadcasted_iota(jnp.int32, (_R, 128), 1))

    tx1 = [targets_ref[0, j, 0] for j in range(_NUM_OBJS)]
    ty1 = [targets_ref[0, j, 1] for j in range(_NUM_OBJS)]
    tx2 = [targets_ref[0, j, 2] for j in range(_NUM_OBJS)]
    ty2 = [targets_ref[0, j, 3] for j in range(_NUM_OBJS)]
    tlab = [targets_ref[0, j, 4] for j in range(_NUM_OBJS)]

    bto = None
    bti = None
    bpi = []
    for j in range(_NUM_OBJS):
        ix = jnp.maximum(jnp.minimum(tx2[j], px2) - jnp.maximum(tx1[j], px1), 0.0)
        iy = jnp.maximum(jnp.minimum(ty2[j], py2) - jnp.maximum(ty1[j], py1), 0.0)
        inter = ix * iy
        area_t = (tx2[j] - tx1[j]) * (ty2[j] - ty1[j])
        ov = inter / (area_t + area_p - inter)
        if j == 0:
            bto = ov
            bti = jnp.zeros((_R, 128), jnp.int32)
        else:
            upd = ov > bto
            bti = jnp.where(upd, jnp.int32(j), bti)
            bto = jnp.where(upd, ov, bto)
        m = jnp.max(ov)
        bpi.append(jnp.min(jnp.where(ov == m, fi, jnp.int32(2 ** 30))))

    for j in range(_NUM_OBJS):
        hit = fi == bpi[j]
        bto = jnp.where(hit, 2.0, bto)
        bti = jnp.where(hit, jnp.int32(j), bti)

    conf_t = jnp.where(bti == 0, tlab[0].astype(jnp.int32), 0)
    mcx = jnp.where(bti == 0, (tx1[0] + tx2[0]) * 0.5, 0.0)
    mcy = jnp.where(bti == 0, (ty1[0] + ty2[0]) * 0.5, 0.0)
    mw = jnp.where(bti == 0, tx2[0] - tx1[0], 0.0)
    mh = jnp.where(bti == 0, ty2[0] - ty1[0], 0.0)
    for j in range(1, _NUM_OBJS):
        sel = bti == j
        conf_t = jnp.where(sel, tlab[j].astype(jnp.int32), conf_t)
        mcx = jnp.where(sel, (tx1[j] + tx2[j]) * 0.5, mcx)
        mcy = jnp.where(sel, (ty1[j] + ty2[j]) * 0.5, mcy)
        mw = jnp.where(sel, tx2[j] - tx1[j], mw)
        mh = jnp.where(sel, ty2[j] - ty1[j], mh)
    conf_t = jnp.where(bto < _THRESHOLD, 0, conf_t)
    pos = conf_t > 0
    posf = pos.astype(f32)
    num_pos = jnp.sum(conf_t > 0, dtype=jnp.int32)

    g_cx = (mcx - pcx) / (_VAR0 * pw)
    g_cy = (mcy - pcy) / (_VAR0 * ph)
    g_w = jnp.log(mw / pw) / _VAR1
    g_h = jnp.log(mh / ph) / _VAR1
    loss_l = jnp.float32(0.0)
    for c, g in enumerate((g_cx, g_cy, g_w, g_h)):
        d = loc_ref[0, c] - g
        ad = jnp.abs(d)
        sl1 = jnp.where(ad < 1.0, 0.5 * d * d, ad - 0.5)
        loss_l = loss_l + jnp.sum(sl1 * posf)

    o0 = obj_ref[0, 0]
    o1 = obj_ref[0, 1]
    mo = jnp.maximum(o0, o1)
    lse_o = mo + jnp.log(jnp.exp(o0 - mo) + jnp.exp(o1 - mo))
    mined_ref[0] = jnp.where(pos, 0.0, lse_o - o0)
    sum_b = jnp.sum(posf * (lse_o - o1))

    cf = [conf_ref[0, c].astype(f32) for c in range(_NUM_CLASSES)]
    s = jnp.exp(cf[0])
    for c in range(1, _NUM_CLASSES):
        s = s + jnp.exp(cf[c])
    lse_c = jnp.log(s)
    c_sel = jnp.zeros((_R, 128), f32)
    for c in range(_NUM_CLASSES):
        c_sel = jnp.where(conf_t == c + 1, cf[c], c_sel)
    sum_a = jnp.sum(posf * (lse_c - c_sel)) + sum_b

    lane = lax.broadcasted_iota(jnp.int32, (1, 8), 1)
    stats_ref[0] = (jnp.where(lane == 0, loss_l, 0.0)
                    + jnp.where(lane == 1, sum_a, 0.0)
                    + jnp.where(lane == 2, sum_b, 0.0)
                    + jnp.where(lane == 3, num_pos.astype(f32), 0.0))


def _mbox_b_kernel(mined_ref, stats_ref, out_ref):
    vals = mined_ref[...]
    keys = lax.bitcast_convert_type(vals, jnp.int32)
    np_f = stats_ref[:, 3:4]
    k = jnp.minimum(jnp.int32(_NEGPOS_RATIO) * np_f.astype(jnp.int32),
                    jnp.int32(_NUM_PRIORS - 1))
    lo = jnp.zeros((_BATCH, 1), jnp.int32)
    for bit in range(30, -1, -1):
        cand = jnp.bitwise_or(lo, jnp.int32(1 << bit))
        cnt = jnp.sum((keys >= cand).astype(jnp.int32), axis=1, keepdims=True)
        lo = jnp.where(cnt >= k, cand, lo)
    gt = keys > lo
    cnt_gt = jnp.sum(gt.astype(jnp.int32), axis=1, keepdims=True)
    sum_gt = jnp.sum(jnp.where(gt, vals, 0.0), axis=1, keepdims=True)
    t_f = lax.bitcast_convert_type(lo, jnp.float32)
    tk = sum_gt + jnp.where(k > 0, (k - cnt_gt).astype(jnp.float32) * t_f, 0.0)

    loss_l = jnp.sum(stats_ref[:, 0])
    loss_c = jnp.sum(stats_ref[:, 1] + tk[:, 0])
    loss_o = jnp.sum(stats_ref[:, 2] + tk[:, 0])
    n = jnp.sum(stats_ref[:, 3])
    lane = lax.broadcasted_iota(jnp.int32, (1, 8), 1)
    out_ref[...] = (jnp.where(lane == 0, loss_l, 0.0)
                    + jnp.where(lane == 1, loss_c, 0.0)
                    + jnp.where(lane == 2, loss_o, 0.0)
                    + jnp.where(lane == 3, n, 0.0))


@jax.jit
def _run(loc_s, conf_s, obj_s, priors_s, targets):
    mined, stats = pl.pallas_call(
        _mbox_a_kernel,
        grid=(_BATCH,),
        in_specs=[
            pl.BlockSpec((4, _R, 128), lambda b: (0, 0, 0)),
            pl.BlockSpec((1, _NUM_OBJS, 5), lambda b: (b, 0, 0)),
            pl.BlockSpec((1, 4, _R, 128), lambda b: (b, 0, 0, 0)),
            pl.BlockSpec((1, _NUM_CLASSES, _R, 128), lambda b: (b, 0, 0, 0)),
            pl.BlockSpec((1, 2, _R, 128), lambda b: (b, 0, 0, 0)),
        ],
        out_specs=[
            pl.BlockSpec((1, _R, 128), lambda b: (b, 0, 0)),
            pl.BlockSpec((1, 1, 8), lambda b: (b, 0, 0)),
        ],
        out_shape=[
            jax.ShapeDtypeStruct((_BATCH, _R, 128), jnp.float32),
            jax.ShapeDtypeStruct((_BATCH, 1, 8), jnp.float32),
        ],
        compiler_params=pltpu.CompilerParams(
            dimension_semantics=("parallel",),
        ),
    )(priors_s, targets, loc_s, conf_s, obj_s)

    out = pl.pallas_call(
        _mbox_b_kernel,
        grid=(1,),
        in_specs=[
            pl.BlockSpec((_BATCH, _NUM_PRIORS), lambda i: (0, 0)),
            pl.BlockSpec((_BATCH, 8), lambda i: (0, 0)),
        ],
        out_specs=pl.BlockSpec((1, 8), lambda i: (0, 0)),
        out_shape=jax.ShapeDtypeStruct((1, 8), jnp.float32),
    )(mined.reshape(_BATCH, _NUM_PRIORS), stats.reshape(_BATCH, 8))
    n = out[0, 3]
    return out[0, 0] / n, out[0, 1] / n, out[0, 2] / n


def kernel(loc_data, conf_data, obj_data, priors, targets):
    loc_s = loc_data.transpose(0, 2, 1).reshape(_BATCH, 4, _R, 128)
    conf_s = (conf_data.astype(jnp.bfloat16)
              .transpose(0, 2, 1).reshape(_BATCH, _NUM_CLASSES, _R, 128))
    obj_s = obj_data.transpose(0, 2, 1).reshape(_BATCH, 2, _R, 128)
    priors_s = priors.T.reshape(4, _R, 128)
    return _run(loc_s, conf_s, obj_s, priors_s, targets)
